# baseline (speedup 1.0000x reference)
import functools

import jax
import jax.numpy as jnp
from jax.experimental import pallas as pl
from jax.experimental.pallas import tpu as pltpu

TEMP = 2.718281828459045
NEG_INF = -1e30


def _moe_body(x_ref, Wg_ref, bg_ref, b1_ref, b2_ref, W1_ref, W2_ref,
              o_ref, w_ref, h_ref, *, na, bt):
    e = pl.program_id(1)
    oh = pl.program_id(2)
    E = Wg_ref.shape[0]

    @pl.when((e == 0) & (oh == 0))
    def _gate():
        x = x_ref[...]
        logits = jax.lax.dot_general(
            x, Wg_ref[...], (((1,), (1,)), ((), ())),
            preferred_element_type=jnp.float32)
        logits_t = jnp.transpose(logits) + bg_ref[...]
        scaled = logits_t / TEMP
        m = jnp.max(scaled, axis=0, keepdims=True)
        ex = jnp.exp(scaled - m)
        p = ex / jnp.sum(ex, axis=0, keepdims=True)
        iota = jax.lax.broadcasted_iota(jnp.int32, (E, bt), 0)
        work = p
        mask = jnp.zeros((E, bt), dtype=jnp.float32)
        for _ in range(na):
            mx = jnp.max(work, axis=0, keepdims=True)
            cand = jnp.where(work == mx, iota, E)
            sel = jnp.min(cand, axis=0, keepdims=True)
            onehot = (iota == sel).astype(jnp.float32)
            mask = mask + onehot
            work = jnp.where(onehot > 0, NEG_INF, work)
        w_t = p * mask
        w_t = w_t / (jnp.sum(w_t, axis=0, keepdims=True) + 1e-8)
        w_ref[...] = jnp.transpose(w_t)

    @pl.when(oh == 0)
    def _hidden():
        x = x_ref[...].astype(jnp.bfloat16)
        h = jax.lax.dot_general(
            x, W1_ref[0].astype(jnp.bfloat16), (((1,), (1,)), ((), ())),
            preferred_element_type=jnp.float32)
        h_ref[...] = jnp.maximum(h + b1_ref[0], 0.0).astype(jnp.bfloat16)

    @pl.when(e == 0)
    def _init():
        o_ref[...] = jax.lax.dot_general(
            w_ref[...], b2_ref[...], (((1,), (0,)), ((), ())),
            preferred_element_type=jnp.float32)

    o = jax.lax.dot_general(
        h_ref[...], W2_ref[0].astype(jnp.bfloat16), (((1,), (1,)), ((), ())),
        preferred_element_type=jnp.float32)
    lane = jax.lax.broadcasted_iota(jnp.int32, (bt, E), 1)
    w_col = jnp.sum(
        jnp.where(lane == e, w_ref[...], 0.0), axis=-1, keepdims=True)
    o_ref[...] += w_col * o


def kernel(x, Wg, bg, W1, b1, W2, b2):
    T, D = x.shape
    E, H, _ = W1.shape
    O = W2.shape[1]
    na = max(1, int(E * 0.7))
    bt = min(2048, T)
    oh = O // 2
    grid = (T // bt, E, 2)

    body = functools.partial(_moe_body, na=na, bt=bt)
    out = pl.pallas_call(
        body,
        grid=grid,
        in_specs=[
            pl.BlockSpec((bt, D), lambda t, e, o2: (t, 0)),
            pl.BlockSpec((E, D), lambda t, e, o2: (0, 0)),
            pl.BlockSpec((E, 1), lambda t, e, o2: (0, 0)),
            pl.BlockSpec((1, 1, H), lambda t, e, o2: (e, 0, 0)),
            pl.BlockSpec((E, oh), lambda t, e, o2: (0, o2)),
            pl.BlockSpec((1, H, D), lambda t, e, o2: (e, 0, 0)),
            pl.BlockSpec((1, oh, H), lambda t, e, o2: (e, o2, 0)),
        ],
        out_specs=pl.BlockSpec((bt, oh), lambda t, e, o2: (t, o2)),
        out_shape=jax.ShapeDtypeStruct((T, O), jnp.float32),
        scratch_shapes=[pltpu.VMEM((bt, E), jnp.float32),
                        pltpu.VMEM((bt, H), jnp.bfloat16)],
        compiler_params=pltpu.CompilerParams(
            dimension_semantics=("parallel", "arbitrary", "arbitrary")),
    )(x, Wg, bg.reshape(E, 1), b1.reshape(E, 1, H), b2, W1, W2)
    return out


# speedup vs baseline: 1.1994x; 1.1994x over previous
import functools

import jax
import jax.numpy as jnp
from jax.experimental import pallas as pl
from jax.experimental.pallas import tpu as pltpu

TEMP = 2.718281828459045
NEG_INF = -1e30


def _moe_body(x_ref, Wg_ref, bg_ref, b1_ref, b2_ref, W1_ref, W2_ref,
              o_ref, w_ref, *, na, bt):
    e = pl.program_id(1)
    hh = pl.program_id(2)
    E = Wg_ref.shape[0]

    @pl.when((e == 0) & (hh == 0))
    def _gate():
        x = x_ref[...]
        logits = jax.lax.dot_general(
            x, Wg_ref[...], (((1,), (1,)), ((), ())),
            preferred_element_type=jnp.float32)
        logits_t = jnp.transpose(logits) + bg_ref[...]
        scaled = logits_t / TEMP
        m = jnp.max(scaled, axis=0, keepdims=True)
        ex = jnp.exp(scaled - m)
        p = ex / jnp.sum(ex, axis=0, keepdims=True)
        iota = jax.lax.broadcasted_iota(jnp.int32, (E, bt), 0)
        work = p
        mask = jnp.zeros((E, bt), dtype=jnp.float32)
        for _ in range(na):
            mx = jnp.max(work, axis=0, keepdims=True)
            cand = jnp.where(work == mx, iota, E)
            sel = jnp.min(cand, axis=0, keepdims=True)
            onehot = (iota == sel).astype(jnp.float32)
            mask = mask + onehot
            work = jnp.where(onehot > 0, NEG_INF, work)
        w_t = p * mask
        w_t = w_t / (jnp.sum(w_t, axis=0, keepdims=True) + 1e-8)
        w = jnp.transpose(w_t)
        w_ref[...] = w
        o_ref[...] = jax.lax.dot_general(
            w, b2_ref[...], (((1,), (0,)), ((), ())),
            preferred_element_type=jnp.float32)

    x = x_ref[...].astype(jnp.bfloat16)
    h = jax.lax.dot_general(
        x, W1_ref[0].astype(jnp.bfloat16), (((1,), (1,)), ((), ())),
        preferred_element_type=jnp.float32)
    h = jnp.maximum(h + b1_ref[0], 0.0).astype(jnp.bfloat16)
    o = jax.lax.dot_general(
        h, W2_ref[0].astype(jnp.bfloat16), (((1,), (1,)), ((), ())),
        preferred_element_type=jnp.float32)
    lane = jax.lax.broadcasted_iota(jnp.int32, (bt, E), 1)
    w_col = jnp.sum(
        jnp.where(lane == e, w_ref[...], 0.0), axis=-1, keepdims=True)
    o_ref[...] += w_col * o


def kernel(x, Wg, bg, W1, b1, W2, b2):
    T, D = x.shape
    E, H, _ = W1.shape
    O = W2.shape[1]
    na = max(1, int(E * 0.7))
    bt = min(2048, T)
    hb = H // 2
    grid = (T // bt, E, 2)

    body = functools.partial(_moe_body, na=na, bt=bt)
    out = pl.pallas_call(
        body,
        grid=grid,
        in_specs=[
            pl.BlockSpec((bt, D), lambda t, e, h2: (t, 0)),
            pl.BlockSpec((E, D), lambda t, e, h2: (0, 0)),
            pl.BlockSpec((E, 1), lambda t, e, h2: (0, 0)),
            pl.BlockSpec((1, 1, hb), lambda t, e, h2: (e, 0, h2)),
            pl.BlockSpec((E, O), lambda t, e, h2: (0, 0)),
            pl.BlockSpec((1, hb, D), lambda t, e, h2: (e, h2, 0)),
            pl.BlockSpec((1, O, hb), lambda t, e, h2: (e, 0, h2)),
        ],
        out_specs=pl.BlockSpec((bt, O), lambda t, e, h2: (t, 0)),
        out_shape=jax.ShapeDtypeStruct((T, O), jnp.float32),
        scratch_shapes=[pltpu.VMEM((bt, E), jnp.float32)],
        compiler_params=pltpu.CompilerParams(
            dimension_semantics=("parallel", "arbitrary", "arbitrary")),
    )(x, Wg, bg.reshape(E, 1), b1.reshape(E, 1, H), b2, W1, W2)
    return out


# speedup vs baseline: 1.2075x; 1.0067x over previous
import functools

import jax
import jax.numpy as jnp
from jax.experimental import pallas as pl
from jax.experimental.pallas import tpu as pltpu

TEMP = 2.718281828459045
NEG_INF = -1e30
H_CHUNKS = 2


def _moe_body(x_ref, Wg_ref, bg_ref, b1_ref, b2_ref, W1_ref, W2_ref,
              o_ref, w_ref, xb_ref, *, na, bt):
    e = pl.program_id(1)
    E = Wg_ref.shape[0]

    @pl.when(e == 0)
    def _gate():
        x = x_ref[...]
        xb_ref[...] = x.astype(jnp.bfloat16)
        logits = jax.lax.dot_general(
            x, Wg_ref[...], (((1,), (1,)), ((), ())),
            preferred_element_type=jnp.float32)
        logits_t = jnp.transpose(logits) + bg_ref[...]
        scaled = logits_t / TEMP
        m = jnp.max(scaled, axis=0, keepdims=True)
        ex = jnp.exp(scaled - m)
        p = ex / jnp.sum(ex, axis=0, keepdims=True)
        iota = jax.lax.broadcasted_iota(jnp.int32, (E, bt), 0)
        work = p
        mask = jnp.zeros((E, bt), dtype=jnp.float32)
        for _ in range(na):
            mx = jnp.max(work, axis=0, keepdims=True)
            cand = jnp.where(work == mx, iota, E)
            sel = jnp.min(cand, axis=0, keepdims=True)
            onehot = (iota == sel).astype(jnp.float32)
            mask = mask + onehot
            work = jnp.where(onehot > 0, NEG_INF, work)
        w_t = p * mask
        w_t = w_t / (jnp.sum(w_t, axis=0, keepdims=True) + 1e-8)
        w = jnp.transpose(w_t)
        w_ref[...] = w
        o_ref[...] = jax.lax.dot_general(
            w, b2_ref[...], (((1,), (0,)), ((), ())),
            preferred_element_type=jnp.float32)

    xb = xb_ref[...]
    H = W1_ref.shape[1]
    hc = H // H_CHUNKS
    o_acc = None
    for k in range(H_CHUNKS):
        w1k = W1_ref[0, k * hc:(k + 1) * hc, :].astype(jnp.bfloat16)
        hk = jax.lax.dot_general(
            xb, w1k, (((1,), (1,)), ((), ())),
            preferred_element_type=jnp.float32)
        hk = jnp.maximum(hk + b1_ref[0, 0, k * hc:(k + 1) * hc], 0.0)
        hk = hk.astype(jnp.bfloat16)
        w2k = W2_ref[0, :, k * hc:(k + 1) * hc].astype(jnp.bfloat16)
        ok = jax.lax.dot_general(
            hk, w2k, (((1,), (1,)), ((), ())),
            preferred_element_type=jnp.float32)
        o_acc = ok if o_acc is None else o_acc + ok
    lane = jax.lax.broadcasted_iota(jnp.int32, (bt, E), 1)
    w_col = jnp.sum(
        jnp.where(lane == e, w_ref[...], 0.0), axis=-1, keepdims=True)
    o_ref[...] += w_col * o_acc


def kernel(x, Wg, bg, W1, b1, W2, b2):
    T, D = x.shape
    E, H, _ = W1.shape
    O = W2.shape[1]
    na = max(1, int(E * 0.7))
    bt = min(1024, T)
    grid = (T // bt, E)

    body = functools.partial(_moe_body, na=na, bt=bt)
    out = pl.pallas_call(
        body,
        grid=grid,
        in_specs=[
            pl.BlockSpec((bt, D), lambda t, e: (t, 0)),
            pl.BlockSpec((E, D), lambda t, e: (0, 0)),
            pl.BlockSpec((E, 1), lambda t, e: (0, 0)),
            pl.BlockSpec((1, 1, H), lambda t, e: (e, 0, 0)),
            pl.BlockSpec((E, O), lambda t, e: (0, 0)),
            pl.BlockSpec((1, H, D), lambda t, e: (e, 0, 0)),
            pl.BlockSpec((1, O, H), lambda t, e: (e, 0, 0)),
        ],
        out_specs=pl.BlockSpec((bt, O), lambda t, e: (t, 0)),
        out_shape=jax.ShapeDtypeStruct((T, O), jnp.float32),
        scratch_shapes=[pltpu.VMEM((bt, E), jnp.float32),
                        pltpu.VMEM((bt, D), jnp.bfloat16)],
        compiler_params=pltpu.CompilerParams(
            dimension_semantics=("parallel", "arbitrary")),
    )(x, Wg, bg.reshape(E, 1), b1.reshape(E, 1, H), b2, W1, W2)
    return out


# speedup vs baseline: 1.2170x; 1.0079x over previous
import functools

import jax
import jax.numpy as jnp
from jax.experimental import pallas as pl
from jax.experimental.pallas import tpu as pltpu

TEMP = 2.718281828459045
NEG_INF = -1e30
H_CHUNKS = 1


def _moe_body(x_ref, Wg_ref, bg_ref, b1_ref, b2_ref, W1_ref, W2_ref,
              o_ref, w_ref, xb_ref, *, na, bt):
    e = pl.program_id(1)
    E = Wg_ref.shape[0]

    @pl.when(e == 0)
    def _gate():
        x = x_ref[...]
        xb_ref[...] = x.astype(jnp.bfloat16)
        logits = jax.lax.dot_general(
            x, Wg_ref[...], (((1,), (1,)), ((), ())),
            preferred_element_type=jnp.float32)
        logits_t = jnp.transpose(logits) + bg_ref[...]
        scaled = logits_t / TEMP
        m = jnp.max(scaled, axis=0, keepdims=True)
        ex = jnp.exp(scaled - m)
        p = ex / jnp.sum(ex, axis=0, keepdims=True)
        iota = jax.lax.broadcasted_iota(jnp.int32, (E, bt), 0)
        work = p
        mask = jnp.zeros((E, bt), dtype=jnp.float32)
        for _ in range(na):
            mx = jnp.max(work, axis=0, keepdims=True)
            cand = jnp.where(work == mx, iota, E)
            sel = jnp.min(cand, axis=0, keepdims=True)
            onehot = (iota == sel).astype(jnp.float32)
            mask = mask + onehot
            work = jnp.where(onehot > 0, NEG_INF, work)
        w_t = p * mask
        w_t = w_t / (jnp.sum(w_t, axis=0, keepdims=True) + 1e-8)
        w = jnp.transpose(w_t)
        w_ref[...] = w
        o_ref[...] = jax.lax.dot_general(
            w, b2_ref[...], (((1,), (0,)), ((), ())),
            preferred_element_type=jnp.float32)

    xb = xb_ref[...]
    H = W1_ref.shape[1]
    hc = H // H_CHUNKS
    o_acc = None
    for k in range(H_CHUNKS):
        w1k = W1_ref[0, k * hc:(k + 1) * hc, :].astype(jnp.bfloat16)
        hk = jax.lax.dot_general(
            xb, w1k, (((1,), (1,)), ((), ())),
            preferred_element_type=jnp.float32)
        hk = jnp.maximum(hk + b1_ref[0, 0, k * hc:(k + 1) * hc], 0.0)
        hk = hk.astype(jnp.bfloat16)
        w2k = W2_ref[0, :, k * hc:(k + 1) * hc].astype(jnp.bfloat16)
        ok = jax.lax.dot_general(
            hk, w2k, (((1,), (1,)), ((), ())),
            preferred_element_type=jnp.float32)
        o_acc = ok if o_acc is None else o_acc + ok
    lane = jax.lax.broadcasted_iota(jnp.int32, (bt, E), 1)
    w_col = jnp.sum(
        jnp.where(lane == e, w_ref[...], 0.0), axis=-1, keepdims=True)
    o_ref[...] += w_col * o_acc


def kernel(x, Wg, bg, W1, b1, W2, b2):
    T, D = x.shape
    E, H, _ = W1.shape
    O = W2.shape[1]
    na = max(1, int(E * 0.7))
    bt = min(1024, T)
    grid = (T // bt, E)

    body = functools.partial(_moe_body, na=na, bt=bt)
    out = pl.pallas_call(
        body,
        grid=grid,
        in_specs=[
            pl.BlockSpec((bt, D), lambda t, e: (t, 0)),
            pl.BlockSpec((E, D), lambda t, e: (0, 0)),
            pl.BlockSpec((E, 1), lambda t, e: (0, 0)),
            pl.BlockSpec((1, 1, H), lambda t, e: (e, 0, 0)),
            pl.BlockSpec((E, O), lambda t, e: (0, 0)),
            pl.BlockSpec((1, H, D), lambda t, e: (e, 0, 0)),
            pl.BlockSpec((1, O, H), lambda t, e: (e, 0, 0)),
        ],
        out_specs=pl.BlockSpec((bt, O), lambda t, e: (t, 0)),
        out_shape=jax.ShapeDtypeStruct((T, O), jnp.float32),
        scratch_shapes=[pltpu.VMEM((bt, E), jnp.float32),
                        pltpu.VMEM((bt, D), jnp.bfloat16)],
        compiler_params=pltpu.CompilerParams(
            dimension_semantics=("parallel", "arbitrary")),
    )(x, Wg, bg.reshape(E, 1), b1.reshape(E, 1, H), b2, W1, W2)
    return out
